# baseline (speedup 1.0000x reference)
import jax
import jax.numpy as jnp
from jax import lax
from jax.experimental import pallas as pl
from jax.experimental.pallas import tpu as pltpu
from jax.experimental.pallas import tpu_sc as plsc

N = 10000
E = 320000
D = 128
DH = D // 2

NC = 2
NS = 16
LANES = 16

EPT_DEG = E // (NC * NS)
K = 80
BLOCKS = E // K
BPT = BLOCKS // NS
NPAD = 10240
RPT = NPAD // NS
ZROWS = 128
NBUF = 4
NSB = 2


def _zero_vmem_1d(ref, n):
    zv = jnp.zeros((LANES,), jnp.float32)

    def body(i, _):
        ref[pl.ds(i * LANES, LANES)] = zv
        return 0

    lax.fori_loop(0, n // LANES, body, 0, unroll=4)


def _zero_vmem_2d(ref, rows, cols):
    zv = jnp.zeros((LANES,), jnp.float32)
    per_row = cols // LANES

    def body(t, _):
        i = t // per_row
        j = t % per_row
        ref[i, pl.ds(j * LANES, LANES)] = zv
        return 0

    lax.fori_loop(0, rows * per_row, body, 0, unroll=4)


def _sc_deg_body(dst_hbm, hists_out, didx, hist):
    c = lax.axis_index("c")
    s = lax.axis_index("s")
    wid = c * NS + s

    _zero_vmem_1d(hist, N)
    pltpu.sync_copy(dst_hbm.at[pl.ds(wid * EPT_DEG, EPT_DEG)], didx)

    ones = jnp.ones((LANES,), jnp.float32)

    def body(j, _):
        idx = didx[pl.ds(j * LANES, LANES)]
        plsc.addupdate_scatter(hist, [idx], ones)
        return 0

    lax.fori_loop(0, EPT_DEG // LANES, body, 0)
    pltpu.sync_copy(hist, hists_out.at[wid])


def _sc_deg(dst):
    mesh = plsc.VectorSubcoreMesh(core_axis_name="c", subcore_axis_name="s")
    f = pl.kernel(
        _sc_deg_body,
        out_type=jax.ShapeDtypeStruct((NC * NS, N), jnp.float32),
        mesh=mesh,
        scratch_types=[
            pltpu.VMEM((EPT_DEG,), jnp.int32),
            pltpu.VMEM((N,), jnp.float32),
        ],
        compiler_params=pltpu.CompilerParams(needs_layout_passes=False),
    )
    return f(dst)


CHUNK = 25
NCHUNK = BPT // CHUNK


def _sc_agg_body(tab_hbm, r2_hbm, src0_hbm, src1_hbm, dst_hbm, ab_out,
                 *rest):
    c = lax.axis_index("c")
    s = lax.axis_index("s")
    it = iter(rest)
    sidx = [next(it), next(it)]
    didx = [next(it), next(it)]
    bufs = [next(it) for _ in range(NBUF)]
    sbufs = [next(it) for _ in range(NSB)]
    sstage = next(it)
    r2v = next(it)
    acca = next(it)
    accb = next(it)
    gsem = [next(it) for _ in range(NBUF)]
    asem = [next(it) for _ in range(NBUF)]
    bsem = [next(it) for _ in range(NSB)]
    isem = [next(it), next(it)]

    _zero_vmem_2d(bufs[0], K, DH)
    for bb in range(RPT // K):
        pltpu.sync_copy(bufs[0], acca.at[pl.ds(s * RPT + bb * K, K)])
        pltpu.sync_copy(bufs[0], accb.at[pl.ds(s * RPT + bb * K, K)])

    base = s * BPT

    def load_chunk(q, p):
        @pl.when(c == 0)
        def _():
            pltpu.make_async_copy(src0_hbm.at[pl.ds(base + q * CHUNK, CHUNK)],
                                  sidx[p], isem[p]).start()

        @pl.when(c == 1)
        def _():
            pltpu.make_async_copy(src1_hbm.at[pl.ds(base + q * CHUNK, CHUNK)],
                                  sidx[p], isem[p]).start()

        pltpu.make_async_copy(dst_hbm.at[pl.ds(base + q * CHUNK, CHUNK)],
                              didx[p], isem[p]).start()

    def wait_chunk(p):
        pltpu.make_async_copy(dst_hbm.at[pl.ds(base, CHUNK)], sidx[p],
                              isem[p]).wait()
        pltpu.make_async_copy(dst_hbm.at[pl.ds(base, CHUNK)], didx[p],
                              isem[p]).wait()

    load_chunk(0, 0)
    wait_chunk(0)
    pltpu.sync_copy(r2_hbm, r2v)
    plsc.subcore_barrier()

    def _bi(j, fn):
        p = (j // CHUNK) % 2

        @pl.when(p == 0)
        def _():
            fn(0)

        @pl.when(p == 1)
        def _():
            fn(1)

    def gather(j, b):
        _bi(j, lambda p: pltpu.make_async_copy(
            tab_hbm.at[sidx[p].at[j % CHUNK]], bufs[b], gsem[b]).start())

    def gwait(j, b):
        pltpu.make_async_copy(tab_hbm.at[sidx[0].at[0]], bufs[b],
                              gsem[b]).wait()

    def scat_a(j, b):
        _bi(j, lambda p: pltpu.make_async_copy(
            bufs[b], acca.at[didx[p].at[j % CHUNK]],
            asem[b]).start(add=True))

    def swait_a(j, b):
        pltpu.make_async_copy(bufs[b], acca.at[didx[0].at[0]], asem[b]).wait()

    def scat_b(j, sb):
        _bi(j, lambda p: pltpu.make_async_copy(
            sbufs[sb], accb.at[didx[p].at[j % CHUNK]],
            bsem[sb]).start(add=True))

    def swait_b(j, sb):
        pltpu.make_async_copy(sbufs[sb], accb.at[didx[0].at[0]],
                              bsem[sb]).wait()

    def scale(j, b, sb):
        def stage(p):
            for g in range(K // LANES):
                sl = pl.ds(g * LANES, LANES)
                sstage[sl] = sidx[p][j % CHUNK, sl] >> 1

        _bi(j, stage)
        for g in range(K // LANES):
            idxv = sstage[pl.ds(g * LANES, LANES)]
            rv = plsc.load_gather(r2v, [idxv])
            for i in range(LANES):
                row = g * LANES + i
                rs = rv[i]
                for q in range(DH // LANES):
                    sl = pl.ds(q * LANES, LANES)
                    sbufs[sb][row, sl] = rs * bufs[b][row, sl]

    gather(0, 0)
    gather(1, 1)

    nfull = BPT // NBUF

    def body(jj, _):
        for b in range(NBUF):
            j = NBUF * jj + b
            sb = b % NSB
            jm = j % CHUNK

            @pl.when(jnp.logical_and(jm == 6, j + (CHUNK - 6) < BPT))
            def _():
                q1 = j // CHUNK + 1

                @pl.when(q1 % 2 == 0)
                def _():
                    load_chunk(q1, 0)

                @pl.when(q1 % 2 == 1)
                def _():
                    load_chunk(q1, 1)

            @pl.when(jm == CHUNK - 2)
            def _():
                q1 = j // CHUNK + 1

                @pl.when(q1 % 2 == 0)
                def _():
                    wait_chunk(0)

                @pl.when(q1 % 2 == 1)
                def _():
                    wait_chunk(1)

            gwait(j, b)
            scat_a(j, b)
            bn = (b + 2) % NBUF

            @pl.when(j + 2 < BPT)
            def _():
                @pl.when(j >= 2)
                def _():
                    swait_a(j - 2, bn)

                gather(j + 2, bn)

            @pl.when(j >= NSB)
            def _():
                swait_b(j - NSB, sb)

            scale(j, b, sb)
            scat_b(j, sb)

        return 0

    lax.fori_loop(0, nfull, body, 0)
    for t in range(nfull * NBUF, BPT):
        b = t % NBUF
        sb = t % NSB
        gwait(t, b)
        scat_a(t, b)
        swait_b(t - NSB, sb)
        scale(t, b, sb)
        scat_b(t, sb)
    for t in range(BPT - NBUF, BPT):
        swait_a(t, t % NBUF)
    for t in range(BPT - NSB, BPT):
        swait_b(t, t % NSB)

    plsc.subcore_barrier()
    for bb in range(RPT // K):
        r0 = s * RPT + bb * K
        pltpu.sync_copy(acca.at[pl.ds(r0, K)], bufs[0])
        pltpu.sync_copy(bufs[0], ab_out.at[0].at[c].at[pl.ds(r0, K)])
        pltpu.sync_copy(accb.at[pl.ds(r0, K)], bufs[1])
        pltpu.sync_copy(bufs[1], ab_out.at[1].at[c].at[pl.ds(r0, K)])


def _sc_agg(tab2, r2, src0_rs, src1_rs, dst_rs):
    mesh = plsc.VectorSubcoreMesh(core_axis_name="c", subcore_axis_name="s")
    f = pl.kernel(
        _sc_agg_body,
        out_type=jax.ShapeDtypeStruct((2, NC, NPAD, DH), jnp.float32),
        mesh=mesh,
        scratch_types=(
            [pltpu.VMEM((CHUNK, K), jnp.int32)] * 2
            + [pltpu.VMEM((CHUNK, K), jnp.int32)] * 2
            + [pltpu.VMEM((K, DH), jnp.float32)] * NBUF
            + [pltpu.VMEM((K, DH), jnp.float32)] * NSB
            + [pltpu.VMEM((K,), jnp.int32),
               pltpu.VMEM((N,), jnp.float32),
               pltpu.VMEM_SHARED((NPAD, DH), jnp.float32),
               pltpu.VMEM_SHARED((NPAD, DH), jnp.float32)]
            + [pltpu.SemaphoreType.DMA] * (2 * NBUF + NSB + 2)
        ),
        compiler_params=pltpu.CompilerParams(needs_layout_passes=False,
                                             use_tc_tiling_on_sc=False),
    )
    return f(tab2, r2, src0_rs, src1_rs, dst_rs)


ROWS_TC = 1000


def _tc_prep_body(hists_ref, r2_ref):
    d = jnp.sum(hists_ref[...], axis=1, keepdims=True)
    d = jnp.maximum(d, 1.0)
    r2_ref[...] = lax.rsqrt(d)


def _tc_prep(hists_t):
    return pl.pallas_call(
        _tc_prep_body,
        grid=(N // ROWS_TC,),
        in_specs=[
            pl.BlockSpec((ROWS_TC, NC * NS), lambda i: (i, 0)),
        ],
        out_specs=pl.BlockSpec((ROWS_TC, 1), lambda i: (i, 0)),
        out_shape=jax.ShapeDtypeStruct((N, 1), jnp.float32),
    )(hists_t)


def _tc_final_body(w_ref, hists_ref, x_ref, alo_ref, ahi_ref, blo_ref,
                   bhi_ref, wgcn_ref, wss_ref, wsn_ref, wgin_ref, wlin_ref,
                   out_ref):
    w0 = w_ref[0]
    w1 = w_ref[1]
    w2 = w_ref[2]
    w3 = w_ref[3]
    d = jnp.sum(hists_ref[...], axis=1, keepdims=True)
    d = jnp.maximum(d, 1.0)
    r = lax.rsqrt(d)
    inv = 1.0 / d
    wmix = w1 * wss_ref[...] + w2 * wgin_ref[...] + w3 * wlin_ref[...]
    x = x_ref[...]
    a = jnp.concatenate([alo_ref[...], ahi_ref[...]], axis=1)
    b = jnp.concatenate([blo_ref[...], bhi_ref[...]], axis=1)
    acc = jnp.dot(x, wmix, preferred_element_type=jnp.float32)
    acc += w2 * jnp.dot(a, wgin_ref[...], preferred_element_type=jnp.float32)
    acc += (w1 * inv) * jnp.dot(a, wsn_ref[...],
                                preferred_element_type=jnp.float32)
    acc += (w0 * r) * jnp.dot(b, wgcn_ref[...],
                              preferred_element_type=jnp.float32)
    out_ref[...] = acc


def _tc_final(weights, hists_t, x, alo, ahi, blo, bhi,
              wgcn, wss, wsn, wgin, wlin):
    row_spec = pl.BlockSpec((ROWS_TC, D), lambda i: (i, 0))
    half_spec = pl.BlockSpec((ROWS_TC, DH), lambda i: (i, 0))
    w_spec = pl.BlockSpec((D, D), lambda i: (0, 0))
    return pl.pallas_call(
        _tc_final_body,
        grid=(N // ROWS_TC,),
        in_specs=[
            pl.BlockSpec(memory_space=pltpu.SMEM),
            pl.BlockSpec((ROWS_TC, NC * NS), lambda i: (i, 0)),
            row_spec,
            half_spec, half_spec, half_spec, half_spec,
            w_spec, w_spec, w_spec, w_spec, w_spec,
        ],
        out_specs=row_spec,
        out_shape=jax.ShapeDtypeStruct((N, D), jnp.float32),
    )(weights, hists_t, x, alo, ahi, blo, bhi, wgcn, wss, wsn, wgin, wlin)


def kernel(x, edge_index, weights, W_gcn, W_sage_self, W_sage_neigh, W_gin,
           W_lin):
    src = edge_index[0]
    dst = edge_index[1]
    src2 = src * 2
    src0_rs = src2.reshape(BLOCKS, K)
    src1_rs = (src2 + 1).reshape(BLOCKS, K)
    dst_rs = dst.reshape(BLOCKS, K)
    x2 = x.reshape(2 * N, DH)

    hists = _sc_deg(dst)
    hists_t = hists.T
    r2 = _tc_prep(hists_t).reshape(N)
    ab = _sc_agg(x2, r2, src0_rs, src1_rs, dst_rs)
    a = ab[0]
    b = ab[1]
    return _tc_final(weights, hists_t, x, a[0], a[1], b[0], b[1],
                     W_gcn, W_sage_self, W_sage_neigh, W_gin, W_lin)


# speedup vs baseline: 1.0939x; 1.0939x over previous
import jax
import jax.numpy as jnp
from jax import lax
from jax.experimental import pallas as pl
from jax.experimental.pallas import tpu as pltpu
from jax.experimental.pallas import tpu_sc as plsc

N = 10000
E = 320000
D = 128
DH = D // 2

NC = 2
NS = 16
LANES = 16

EPT_DEG = E // (NC * NS)
K = 80
BLOCKS = E // K
BPT = BLOCKS // NS
NPAD = 10240
RPT = NPAD // NS
ZROWS = 128
NBUF = 4
NSB = 2


def _zero_vmem_1d(ref, n):
    zv = jnp.zeros((LANES,), jnp.float32)

    def body(i, _):
        ref[pl.ds(i * LANES, LANES)] = zv
        return 0

    lax.fori_loop(0, n // LANES, body, 0, unroll=4)


def _zero_vmem_2d(ref, rows, cols):
    zv = jnp.zeros((LANES,), jnp.float32)
    per_row = cols // LANES

    def body(t, _):
        i = t // per_row
        j = t % per_row
        ref[i, pl.ds(j * LANES, LANES)] = zv
        return 0

    lax.fori_loop(0, rows * per_row, body, 0, unroll=4)


def _sc_deg_body(dst_hbm, hists_out, didx, hist):
    c = lax.axis_index("c")
    s = lax.axis_index("s")
    wid = c * NS + s

    _zero_vmem_1d(hist, N)
    pltpu.sync_copy(dst_hbm.at[pl.ds(wid * EPT_DEG, EPT_DEG)], didx)

    ones = jnp.ones((LANES,), jnp.float32)

    def body(j, _):
        idx = didx[pl.ds(j * LANES, LANES)]
        plsc.addupdate_scatter(hist, [idx], ones)
        return 0

    lax.fori_loop(0, EPT_DEG // LANES, body, 0)
    pltpu.sync_copy(hist, hists_out.at[wid])


def _sc_deg(dst):
    mesh = plsc.VectorSubcoreMesh(core_axis_name="c", subcore_axis_name="s")
    f = pl.kernel(
        _sc_deg_body,
        out_type=jax.ShapeDtypeStruct((NC * NS, N), jnp.float32),
        mesh=mesh,
        scratch_types=[
            pltpu.VMEM((EPT_DEG,), jnp.int32),
            pltpu.VMEM((N,), jnp.float32),
        ],
        compiler_params=pltpu.CompilerParams(needs_layout_passes=False),
    )
    return f(dst)


CHUNK = 25
NCHUNK = BPT // CHUNK


def _sc_agg_body(tab_hbm, r2_hbm, src0_hbm, src1_hbm, dst_hbm, ab_out,
                 *rest):
    c = lax.axis_index("c")
    s = lax.axis_index("s")
    it = iter(rest)
    sidx = [next(it), next(it)]
    didx = [next(it), next(it)]
    bufs = [next(it) for _ in range(NBUF)]
    sbufs = [next(it) for _ in range(NSB)]
    sstage = next(it)
    r2v = next(it)
    acca = next(it)
    accb = next(it)
    gsem = [next(it) for _ in range(NBUF)]
    asem = [next(it) for _ in range(NBUF)]
    bsem = [next(it) for _ in range(NSB)]
    isem = [next(it), next(it)]

    _zero_vmem_2d(bufs[0], K, DH)
    for bb in range(RPT // K):
        pltpu.sync_copy(bufs[0], acca.at[pl.ds(s * RPT + bb * K, K)])
        pltpu.sync_copy(bufs[0], accb.at[pl.ds(s * RPT + bb * K, K)])

    base = s * BPT

    def load_chunk(q, p):
        @pl.when(c == 0)
        def _():
            pltpu.make_async_copy(src0_hbm.at[pl.ds(base + q * CHUNK, CHUNK)],
                                  sidx[p], isem[p]).start()

        @pl.when(c == 1)
        def _():
            pltpu.make_async_copy(src1_hbm.at[pl.ds(base + q * CHUNK, CHUNK)],
                                  sidx[p], isem[p]).start()

        pltpu.make_async_copy(dst_hbm.at[pl.ds(base + q * CHUNK, CHUNK)],
                              didx[p], isem[p]).start()

    def wait_chunk(p):
        pltpu.make_async_copy(dst_hbm.at[pl.ds(base, CHUNK)], sidx[p],
                              isem[p]).wait()
        pltpu.make_async_copy(dst_hbm.at[pl.ds(base, CHUNK)], didx[p],
                              isem[p]).wait()

    load_chunk(0, 0)
    wait_chunk(0)
    pltpu.sync_copy(r2_hbm, r2v)
    plsc.subcore_barrier()

    def _bi(j, fn):
        p = (j // CHUNK) % 2

        @pl.when(p == 0)
        def _():
            fn(0)

        @pl.when(p == 1)
        def _():
            fn(1)

    def gather(j, b):
        _bi(j, lambda p: pltpu.make_async_copy(
            tab_hbm.at[sidx[p].at[j % CHUNK]], bufs[b], gsem[b]).start())

    def gwait(j, b):
        pltpu.make_async_copy(tab_hbm.at[sidx[0].at[0]], bufs[b],
                              gsem[b]).wait()

    def scat_a(j, b):
        _bi(j, lambda p: pltpu.make_async_copy(
            bufs[b], acca.at[didx[p].at[j % CHUNK]],
            asem[b]).start(add=True))

    def swait_a(j, b):
        pltpu.make_async_copy(bufs[b], acca.at[didx[0].at[0]], asem[b]).wait()

    def scat_b(j, sb):
        _bi(j, lambda p: pltpu.make_async_copy(
            sbufs[sb], accb.at[didx[p].at[j % CHUNK]],
            bsem[sb]).start(add=True))

    def swait_b(j, sb):
        pltpu.make_async_copy(sbufs[sb], accb.at[didx[0].at[0]],
                              bsem[sb]).wait()

    def scale(j, b, sb):
        def stage(p):
            for g in range(K // LANES):
                sl = pl.ds(g * LANES, LANES)
                sstage[sl] = sidx[p][j % CHUNK, sl] >> 1

        _bi(j, stage)
        for g in range(K // LANES):
            idxv = sstage[pl.ds(g * LANES, LANES)]
            rv = plsc.load_gather(r2v, [idxv])
            for i in range(LANES):
                row = g * LANES + i
                rs = rv[i]
                for q in range(DH // LANES):
                    sl = pl.ds(q * LANES, LANES)
                    sbufs[sb][row, sl] = rs * bufs[b][row, sl]

    gather(0, 0)
    gather(1, 1)

    nfull = BPT // NBUF

    def body(jj, _):
        for b in range(NBUF):
            j = NBUF * jj + b
            sb = b % NSB
            jm = j % CHUNK

            @pl.when(jnp.logical_and(jm == 6, j + (CHUNK - 6) < BPT))
            def _():
                q1 = j // CHUNK + 1

                @pl.when(q1 % 2 == 0)
                def _():
                    load_chunk(q1, 0)

                @pl.when(q1 % 2 == 1)
                def _():
                    load_chunk(q1, 1)

            @pl.when(jm == CHUNK - 2)
            def _():
                q1 = j // CHUNK + 1

                @pl.when(q1 % 2 == 0)
                def _():
                    wait_chunk(0)

                @pl.when(q1 % 2 == 1)
                def _():
                    wait_chunk(1)

            gwait(j, b)
            scat_a(j, b)
            bn = (b + 2) % NBUF

            @pl.when(j + 2 < BPT)
            def _():
                @pl.when(j >= 2)
                def _():
                    swait_a(j - 2, bn)

                gather(j + 2, bn)

            @pl.when(j >= NSB)
            def _():
                swait_b(j - NSB, sb)

            scale(j, b, sb)
            scat_b(j, sb)

        return 0

    lax.fori_loop(0, nfull, body, 0)
    for t in range(nfull * NBUF, BPT):
        b = t % NBUF
        sb = t % NSB
        gwait(t, b)
        scat_a(t, b)
        swait_b(t - NSB, sb)
        scale(t, b, sb)
        scat_b(t, sb)
    for t in range(BPT - NBUF, BPT):
        swait_a(t, t % NBUF)
    for t in range(BPT - NSB, BPT):
        swait_b(t, t % NSB)

    plsc.subcore_barrier()
    for bb in range(RPT // K):
        r0 = s * RPT + bb * K
        pltpu.sync_copy(acca.at[pl.ds(r0, K)], bufs[0])
        pltpu.sync_copy(bufs[0], ab_out.at[0].at[c].at[pl.ds(r0, K)])
        pltpu.sync_copy(accb.at[pl.ds(r0, K)], bufs[1])
        pltpu.sync_copy(bufs[1], ab_out.at[1].at[c].at[pl.ds(r0, K)])


def _sc_agg(tab2, r2, src0_rs, src1_rs, dst_rs):
    mesh = plsc.VectorSubcoreMesh(core_axis_name="c", subcore_axis_name="s")
    f = pl.kernel(
        _sc_agg_body,
        out_type=jax.ShapeDtypeStruct((2, NC, NPAD, DH), jnp.float32),
        mesh=mesh,
        scratch_types=(
            [pltpu.VMEM((CHUNK, K), jnp.int32)] * 2
            + [pltpu.VMEM((CHUNK, K), jnp.int32)] * 2
            + [pltpu.VMEM((K, DH), jnp.float32)] * NBUF
            + [pltpu.VMEM((K, DH), jnp.float32)] * NSB
            + [pltpu.VMEM((K,), jnp.int32),
               pltpu.VMEM((N,), jnp.float32),
               pltpu.VMEM_SHARED((NPAD, DH), jnp.float32),
               pltpu.VMEM_SHARED((NPAD, DH), jnp.float32)]
            + [pltpu.SemaphoreType.DMA] * (2 * NBUF + NSB + 2)
        ),
        compiler_params=pltpu.CompilerParams(needs_layout_passes=False,
                                             use_tc_tiling_on_sc=False),
    )
    return f(tab2, r2, src0_rs, src1_rs, dst_rs)


ROWS_TC = 1000


def _tc_prep_body(hists_ref, r2_ref):
    d = jnp.sum(hists_ref[...], axis=1, keepdims=True)
    d = jnp.maximum(d, 1.0)
    r2_ref[...] = lax.rsqrt(d)


def _tc_prep(hists_t):
    return pl.pallas_call(
        _tc_prep_body,
        grid=(N // ROWS_TC,),
        in_specs=[
            pl.BlockSpec((ROWS_TC, NC * NS), lambda i: (i, 0)),
        ],
        out_specs=pl.BlockSpec((ROWS_TC, 1), lambda i: (i, 0)),
        out_shape=jax.ShapeDtypeStruct((N, 1), jnp.float32),
    )(hists_t)


def _tc_final_body(w_ref, hists_ref, x_ref, alo_ref, ahi_ref, blo_ref,
                   bhi_ref, wgcn_ref, wss_ref, wsn_ref, wgin_ref, wlin_ref,
                   out_ref):
    w0 = w_ref[0]
    w1 = w_ref[1]
    w2 = w_ref[2]
    w3 = w_ref[3]
    d = jnp.sum(hists_ref[...], axis=1, keepdims=True)
    d = jnp.maximum(d, 1.0)
    r = lax.rsqrt(d)
    inv = 1.0 / d
    wmix = w1 * wss_ref[...] + w2 * wgin_ref[...] + w3 * wlin_ref[...]
    x = x_ref[...]
    a = jnp.concatenate([alo_ref[0, 0], ahi_ref[0, 0]], axis=1)
    b = jnp.concatenate([blo_ref[0, 0], bhi_ref[0, 0]], axis=1)
    acc = jnp.dot(x, wmix, preferred_element_type=jnp.float32)
    acc += w2 * jnp.dot(a, wgin_ref[...], preferred_element_type=jnp.float32)
    acc += (w1 * inv) * jnp.dot(a, wsn_ref[...],
                                preferred_element_type=jnp.float32)
    acc += (w0 * r) * jnp.dot(b, wgcn_ref[...],
                              preferred_element_type=jnp.float32)
    out_ref[...] = acc


def _tc_final(weights, hists_t, x, ab, wgcn, wss, wsn, wgin, wlin):
    row_spec = pl.BlockSpec((ROWS_TC, D), lambda i: (i, 0))
    w_spec = pl.BlockSpec((D, D), lambda i: (0, 0))

    def quarter(m, c):
        return pl.BlockSpec((1, 1, ROWS_TC, DH), lambda i, m=m, c=c: (m, c, i, 0))

    return pl.pallas_call(
        _tc_final_body,
        grid=(N // ROWS_TC,),
        in_specs=[
            pl.BlockSpec(memory_space=pltpu.SMEM),
            pl.BlockSpec((ROWS_TC, NC * NS), lambda i: (i, 0)),
            row_spec,
            quarter(0, 0), quarter(0, 1), quarter(1, 0), quarter(1, 1),
            w_spec, w_spec, w_spec, w_spec, w_spec,
        ],
        out_specs=row_spec,
        out_shape=jax.ShapeDtypeStruct((N, D), jnp.float32),
    )(weights, hists_t, x, ab, ab, ab, ab, wgcn, wss, wsn, wgin, wlin)


def kernel(x, edge_index, weights, W_gcn, W_sage_self, W_sage_neigh, W_gin,
           W_lin):
    src = edge_index[0]
    dst = edge_index[1]
    src2 = src * 2
    src0_rs = src2.reshape(BLOCKS, K)
    src1_rs = (src2 + 1).reshape(BLOCKS, K)
    dst_rs = dst.reshape(BLOCKS, K)
    x2 = x.reshape(2 * N, DH)

    hists = _sc_deg(dst)
    hists_t = hists.T
    r2 = _tc_prep(hists_t).reshape(N)
    ab = _sc_agg(x2, r2, src0_rs, src1_rs, dst_rs)
    return _tc_final(weights, hists_t, x, ab,
                     W_gcn, W_sage_self, W_sage_neigh, W_gin, W_lin)


# speedup vs baseline: 1.1011x; 1.0066x over previous
import jax
import jax.numpy as jnp
from jax import lax
from jax.experimental import pallas as pl
from jax.experimental.pallas import tpu as pltpu
from jax.experimental.pallas import tpu_sc as plsc

N = 10000
E = 320000
D = 128
DH = D // 2

NC = 2
NS = 16
LANES = 16

EPT_DEG = E // (NC * NS)
K = 80
BLOCKS = E // K
BPT = BLOCKS // NS
NPAD = 10240
RPT = NPAD // NS
ZROWS = 128
NBUF = 4
NSB = 2


def _zero_vmem_1d(ref, n):
    zv = jnp.zeros((LANES,), jnp.float32)

    def body(i, _):
        ref[pl.ds(i * LANES, LANES)] = zv
        return 0

    lax.fori_loop(0, n // LANES, body, 0, unroll=4)


def _zero_vmem_2d(ref, rows, cols):
    zv = jnp.zeros((LANES,), jnp.float32)
    per_row = cols // LANES

    def body(t, _):
        i = t // per_row
        j = t % per_row
        ref[i, pl.ds(j * LANES, LANES)] = zv
        return 0

    lax.fori_loop(0, rows * per_row, body, 0, unroll=4)


def _sc_deg_body(dst_hbm, hists_out, didx, hist):
    c = lax.axis_index("c")
    s = lax.axis_index("s")
    wid = c * NS + s

    _zero_vmem_1d(hist, N)
    pltpu.sync_copy(dst_hbm.at[pl.ds(wid * EPT_DEG, EPT_DEG)], didx)

    ones = jnp.ones((LANES,), jnp.float32)

    def body(j, _):
        idx = didx[pl.ds(j * LANES, LANES)]
        plsc.addupdate_scatter(hist, [idx], ones)
        return 0

    lax.fori_loop(0, EPT_DEG // LANES, body, 0)
    pltpu.sync_copy(hist, hists_out.at[wid])


def _sc_deg(dst):
    mesh = plsc.VectorSubcoreMesh(core_axis_name="c", subcore_axis_name="s")
    f = pl.kernel(
        _sc_deg_body,
        out_type=jax.ShapeDtypeStruct((NC * NS, N), jnp.float32),
        mesh=mesh,
        scratch_types=[
            pltpu.VMEM((EPT_DEG,), jnp.int32),
            pltpu.VMEM((N,), jnp.float32),
        ],
        compiler_params=pltpu.CompilerParams(needs_layout_passes=False),
    )
    return f(dst)


CHUNK = 25
NCHUNK = BPT // CHUNK


def _sc_agg_body(tab_hbm, r2_hbm, src0_hbm, src1_hbm, dst_hbm, ab_out,
                 *rest):
    c = lax.axis_index("c")
    s = lax.axis_index("s")
    it = iter(rest)
    sidx = [next(it), next(it)]
    didx = [next(it), next(it)]
    bufs = [next(it) for _ in range(NBUF)]
    sbufs = [next(it) for _ in range(NSB)]
    sstage = next(it)
    r2v = next(it)
    acca = next(it)
    accb = next(it)
    gsem = [next(it) for _ in range(NBUF)]
    asem = [next(it) for _ in range(NBUF)]
    bsem = [next(it) for _ in range(NSB)]
    isem = [next(it), next(it)]

    _zero_vmem_2d(bufs[0], K, DH)
    for bb in range(RPT // K):
        pltpu.sync_copy(bufs[0], acca.at[pl.ds(s * RPT + bb * K, K)])
        pltpu.sync_copy(bufs[0], accb.at[pl.ds(s * RPT + bb * K, K)])

    base = s * BPT

    def load_chunk(q, p):
        @pl.when(c == 0)
        def _():
            pltpu.make_async_copy(src0_hbm.at[pl.ds(base + q * CHUNK, CHUNK)],
                                  sidx[p], isem[p]).start()

        @pl.when(c == 1)
        def _():
            pltpu.make_async_copy(src1_hbm.at[pl.ds(base + q * CHUNK, CHUNK)],
                                  sidx[p], isem[p]).start()

        pltpu.make_async_copy(dst_hbm.at[pl.ds(base + q * CHUNK, CHUNK)],
                              didx[p], isem[p]).start()

    def wait_chunk(p):
        pltpu.make_async_copy(dst_hbm.at[pl.ds(base, CHUNK)], sidx[p],
                              isem[p]).wait()
        pltpu.make_async_copy(dst_hbm.at[pl.ds(base, CHUNK)], didx[p],
                              isem[p]).wait()

    load_chunk(0, 0)
    wait_chunk(0)
    pltpu.sync_copy(r2_hbm, r2v)
    plsc.subcore_barrier()

    def _bi(j, fn):
        p = (j // CHUNK) % 2

        @pl.when(p == 0)
        def _():
            fn(0)

        @pl.when(p == 1)
        def _():
            fn(1)

    def gather(j, b):
        _bi(j, lambda p: pltpu.make_async_copy(
            tab_hbm.at[sidx[p].at[j % CHUNK]], bufs[b], gsem[b]).start())

    def gwait(j, b):
        pltpu.make_async_copy(tab_hbm.at[sidx[0].at[0]], bufs[b],
                              gsem[b]).wait()

    def scat_a(j, b):
        _bi(j, lambda p: pltpu.make_async_copy(
            bufs[b], acca.at[didx[p].at[j % CHUNK]],
            asem[b]).start(add=True))

    def swait_a(j, b):
        pltpu.make_async_copy(bufs[b], acca.at[didx[0].at[0]], asem[b]).wait()

    def scat_b(j, sb):
        _bi(j, lambda p: pltpu.make_async_copy(
            sbufs[sb], accb.at[didx[p].at[j % CHUNK]],
            bsem[sb]).start(add=True))

    def swait_b(j, sb):
        pltpu.make_async_copy(sbufs[sb], accb.at[didx[0].at[0]],
                              bsem[sb]).wait()

    def scale(j, b, sb):
        def stage(p):
            for g in range(K // LANES):
                sl = pl.ds(g * LANES, LANES)
                sstage[sl] = sidx[p][j % CHUNK, sl] >> 1

        _bi(j, stage)
        for g in range(K // LANES):
            idxv = sstage[pl.ds(g * LANES, LANES)]
            rv = plsc.load_gather(r2v, [idxv])
            for i in range(LANES):
                row = g * LANES + i
                rs = rv[i]
                for q in range(DH // LANES):
                    sl = pl.ds(q * LANES, LANES)
                    sbufs[sb][row, sl] = rs * bufs[b][row, sl]

    gather(0, 0)
    gather(1, 1)

    nfull = BPT // NBUF

    def body(jj, _):
        for b in range(NBUF):
            j = NBUF * jj + b
            sb = b % NSB
            jm = j % CHUNK

            @pl.when(jnp.logical_and(jm == 6, j + (CHUNK - 6) < BPT))
            def _():
                q1 = j // CHUNK + 1

                @pl.when(q1 % 2 == 0)
                def _():
                    load_chunk(q1, 0)

                @pl.when(q1 % 2 == 1)
                def _():
                    load_chunk(q1, 1)

            @pl.when(jm == CHUNK - 2)
            def _():
                q1 = j // CHUNK + 1

                @pl.when(q1 % 2 == 0)
                def _():
                    wait_chunk(0)

                @pl.when(q1 % 2 == 1)
                def _():
                    wait_chunk(1)

            gwait(j, b)
            scat_a(j, b)
            bn = (b + 2) % NBUF

            @pl.when(j + 2 < BPT)
            def _():
                @pl.when(j >= 2)
                def _():
                    swait_a(j - 2, bn)

                gather(j + 2, bn)

            @pl.when(j >= NSB)
            def _():
                swait_b(j - NSB, sb)

            scale(j, b, sb)
            scat_b(j, sb)

        return 0

    lax.fori_loop(0, nfull, body, 0)
    for t in range(nfull * NBUF, BPT):
        b = t % NBUF
        sb = t % NSB
        gwait(t, b)
        scat_a(t, b)
        swait_b(t - NSB, sb)
        scale(t, b, sb)
        scat_b(t, sb)
    for t in range(BPT - NBUF, BPT):
        swait_a(t, t % NBUF)
    for t in range(BPT - NSB, BPT):
        swait_b(t, t % NSB)

    plsc.subcore_barrier()
    r0 = s * RPT
    pltpu.make_async_copy(acca.at[pl.ds(r0, RPT)],
                          ab_out.at[0].at[c].at[pl.ds(r0, RPT)],
                          gsem[0]).start()
    pltpu.make_async_copy(accb.at[pl.ds(r0, RPT)],
                          ab_out.at[1].at[c].at[pl.ds(r0, RPT)],
                          gsem[1]).start()
    pltpu.make_async_copy(acca.at[pl.ds(r0, RPT)],
                          ab_out.at[0].at[c].at[pl.ds(r0, RPT)],
                          gsem[0]).wait()
    pltpu.make_async_copy(accb.at[pl.ds(r0, RPT)],
                          ab_out.at[1].at[c].at[pl.ds(r0, RPT)],
                          gsem[1]).wait()


def _sc_agg(tab2, r2, src0_rs, src1_rs, dst_rs):
    mesh = plsc.VectorSubcoreMesh(core_axis_name="c", subcore_axis_name="s")
    f = pl.kernel(
        _sc_agg_body,
        out_type=jax.ShapeDtypeStruct((2, NC, NPAD, DH), jnp.float32),
        mesh=mesh,
        scratch_types=(
            [pltpu.VMEM((CHUNK, K), jnp.int32)] * 2
            + [pltpu.VMEM((CHUNK, K), jnp.int32)] * 2
            + [pltpu.VMEM((K, DH), jnp.float32)] * NBUF
            + [pltpu.VMEM((K, DH), jnp.float32)] * NSB
            + [pltpu.VMEM((K,), jnp.int32),
               pltpu.VMEM((N,), jnp.float32),
               pltpu.VMEM_SHARED((NPAD, DH), jnp.float32),
               pltpu.VMEM_SHARED((NPAD, DH), jnp.float32)]
            + [pltpu.SemaphoreType.DMA] * (2 * NBUF + NSB + 2)
        ),
        compiler_params=pltpu.CompilerParams(needs_layout_passes=False,
                                             use_tc_tiling_on_sc=False),
    )
    return f(tab2, r2, src0_rs, src1_rs, dst_rs)


ROWS_TC = 1000


def _tc_prep_body(hists_ref, r2_ref):
    d = jnp.sum(hists_ref[...], axis=1, keepdims=True)
    d = jnp.maximum(d, 1.0)
    r2_ref[...] = lax.rsqrt(d)


def _tc_prep(hists_t):
    return pl.pallas_call(
        _tc_prep_body,
        grid=(N // ROWS_TC,),
        in_specs=[
            pl.BlockSpec((ROWS_TC, NC * NS), lambda i: (i, 0)),
        ],
        out_specs=pl.BlockSpec((ROWS_TC, 1), lambda i: (i, 0)),
        out_shape=jax.ShapeDtypeStruct((N, 1), jnp.float32),
    )(hists_t)


def _tc_final_body(w_ref, hists_ref, x_ref, alo_ref, ahi_ref, blo_ref,
                   bhi_ref, wgcn_ref, wss_ref, wsn_ref, wgin_ref, wlin_ref,
                   out_ref):
    w0 = w_ref[0]
    w1 = w_ref[1]
    w2 = w_ref[2]
    w3 = w_ref[3]
    d = jnp.sum(hists_ref[...], axis=1, keepdims=True)
    d = jnp.maximum(d, 1.0)
    r = lax.rsqrt(d)
    inv = 1.0 / d
    wmix = w1 * wss_ref[...] + w2 * wgin_ref[...] + w3 * wlin_ref[...]
    x = x_ref[...]
    a = jnp.concatenate([alo_ref[0, 0], ahi_ref[0, 0]], axis=1)
    b = jnp.concatenate([blo_ref[0, 0], bhi_ref[0, 0]], axis=1)
    acc = jnp.dot(x, wmix, preferred_element_type=jnp.float32)
    acc += w2 * jnp.dot(a, wgin_ref[...], preferred_element_type=jnp.float32)
    acc += (w1 * inv) * jnp.dot(a, wsn_ref[...],
                                preferred_element_type=jnp.float32)
    acc += (w0 * r) * jnp.dot(b, wgcn_ref[...],
                              preferred_element_type=jnp.float32)
    out_ref[...] = acc


def _tc_final(weights, hists_t, x, ab, wgcn, wss, wsn, wgin, wlin):
    row_spec = pl.BlockSpec((ROWS_TC, D), lambda i: (i, 0))
    w_spec = pl.BlockSpec((D, D), lambda i: (0, 0))

    def quarter(m, c):
        return pl.BlockSpec((1, 1, ROWS_TC, DH), lambda i, m=m, c=c: (m, c, i, 0))

    return pl.pallas_call(
        _tc_final_body,
        grid=(N // ROWS_TC,),
        in_specs=[
            pl.BlockSpec(memory_space=pltpu.SMEM),
            pl.BlockSpec((ROWS_TC, NC * NS), lambda i: (i, 0)),
            row_spec,
            quarter(0, 0), quarter(0, 1), quarter(1, 0), quarter(1, 1),
            w_spec, w_spec, w_spec, w_spec, w_spec,
        ],
        out_specs=row_spec,
        out_shape=jax.ShapeDtypeStruct((N, D), jnp.float32),
    )(weights, hists_t, x, ab, ab, ab, ab, wgcn, wss, wsn, wgin, wlin)


def kernel(x, edge_index, weights, W_gcn, W_sage_self, W_sage_neigh, W_gin,
           W_lin):
    src = edge_index[0]
    dst = edge_index[1]
    src2 = src * 2
    src0_rs = src2.reshape(BLOCKS, K)
    src1_rs = (src2 + 1).reshape(BLOCKS, K)
    dst_rs = dst.reshape(BLOCKS, K)
    x2 = x.reshape(2 * N, DH)

    hists = _sc_deg(dst)
    hists_t = hists.T
    r2 = _tc_prep(hists_t).reshape(N)
    ab = _sc_agg(x2, r2, src0_rs, src1_rs, dst_rs)
    return _tc_final(weights, hists_t, x, ab,
                     W_gcn, W_sage_self, W_sage_neigh, W_gin, W_lin)


# speedup vs baseline: 1.1075x; 1.0059x over previous
import jax
import jax.numpy as jnp
from jax import lax
from jax.experimental import pallas as pl
from jax.experimental.pallas import tpu as pltpu
from jax.experimental.pallas import tpu_sc as plsc

N = 10000
E = 320000
D = 128
DH = D // 2

NC = 2
NS = 16
LANES = 16

EPT_DEG = E // (NC * NS)
K = 80
BLOCKS = E // K
BPT = BLOCKS // NS
NPAD = 10240
RPT = NPAD // NS
NBUF = 4
NSB = 2


def _zero_vmem_1d(ref, n):
    zv = jnp.zeros((LANES,), jnp.float32)

    def body(i, _):
        ref[pl.ds(i * LANES, LANES)] = zv
        return 0

    lax.fori_loop(0, n // LANES, body, 0, unroll=4)


def _zero_vmem_2d(ref, rows, cols):
    zv = jnp.zeros((LANES,), jnp.float32)
    per_row = cols // LANES

    def body(t, _):
        i = t // per_row
        j = t % per_row
        ref[i, pl.ds(j * LANES, LANES)] = zv
        return 0

    lax.fori_loop(0, rows * per_row, body, 0, unroll=4)


def _sc_deg_body(dst_hbm, hists_out, didx, hist):
    c = lax.axis_index("c")
    s = lax.axis_index("s")
    wid = c * NS + s

    _zero_vmem_1d(hist, N)
    pltpu.sync_copy(dst_hbm.at[pl.ds(wid * EPT_DEG, EPT_DEG)], didx)

    ones = jnp.ones((LANES,), jnp.float32)

    def body(j, _):
        idx = didx[pl.ds(j * LANES, LANES)]
        plsc.addupdate_scatter(hist, [idx], ones)
        return 0

    lax.fori_loop(0, EPT_DEG // LANES, body, 0)
    pltpu.sync_copy(hist, hists_out.at[wid])


def _sc_deg(dst):
    mesh = plsc.VectorSubcoreMesh(core_axis_name="c", subcore_axis_name="s")
    f = pl.kernel(
        _sc_deg_body,
        out_type=jax.ShapeDtypeStruct((NC * NS, N), jnp.float32),
        mesh=mesh,
        scratch_types=[
            pltpu.VMEM((EPT_DEG,), jnp.int32),
            pltpu.VMEM((N,), jnp.float32),
        ],
        compiler_params=pltpu.CompilerParams(needs_layout_passes=False),
    )
    return f(dst)


CHUNK = 25


def _sc_agg_body(tab_hbm, r2_hbm, src0_hbm, src1_hbm, dst_hbm, ab_out,
                 *rest):
    c = lax.axis_index("c")
    s = lax.axis_index("s")
    it = iter(rest)
    sidx = [next(it), next(it)]
    didx = [next(it), next(it)]
    bufs = [next(it) for _ in range(NBUF)]
    sbufs = [next(it) for _ in range(NSB)]
    sstage = next(it)
    r2v = next(it)
    acca = next(it)
    accb = next(it)
    gsem = [next(it) for _ in range(NBUF)]
    asem = [next(it) for _ in range(NBUF)]
    bsem = [next(it) for _ in range(NSB)]
    isem = [next(it), next(it)]

    _zero_vmem_2d(bufs[0], K, DH)
    for bb in range(RPT // K):
        pltpu.sync_copy(bufs[0], acca.at[pl.ds(s * RPT + bb * K, K)])
        pltpu.sync_copy(bufs[0], accb.at[pl.ds(s * RPT + bb * K, K)])

    base = s * BPT

    def load_chunk(q, p):
        @pl.when(c == 0)
        def _():
            pltpu.make_async_copy(src0_hbm.at[pl.ds(base + q * CHUNK, CHUNK)],
                                  sidx[p], isem[p]).start()

        @pl.when(c == 1)
        def _():
            pltpu.make_async_copy(src1_hbm.at[pl.ds(base + q * CHUNK, CHUNK)],
                                  sidx[p], isem[p]).start()

        pltpu.make_async_copy(dst_hbm.at[pl.ds(base + q * CHUNK, CHUNK)],
                              didx[p], isem[p]).start()

    def wait_chunk(p):
        pltpu.make_async_copy(dst_hbm.at[pl.ds(base, CHUNK)], sidx[p],
                              isem[p]).wait()
        pltpu.make_async_copy(dst_hbm.at[pl.ds(base, CHUNK)], didx[p],
                              isem[p]).wait()

    load_chunk(0, 0)
    wait_chunk(0)
    pltpu.sync_copy(r2_hbm, r2v)
    plsc.subcore_barrier()

    def _bi(j, fn):
        p = (j // CHUNK) % 2

        @pl.when(p == 0)
        def _():
            fn(0)

        @pl.when(p == 1)
        def _():
            fn(1)

    def gather(j, b):
        _bi(j, lambda p: pltpu.make_async_copy(
            tab_hbm.at[sidx[p].at[j % CHUNK]], bufs[b], gsem[b]).start())

    def gwait(j, b):
        pltpu.make_async_copy(tab_hbm.at[sidx[0].at[0]], bufs[b],
                              gsem[b]).wait()

    def scat_a(j, b):
        _bi(j, lambda p: pltpu.make_async_copy(
            bufs[b], acca.at[didx[p].at[j % CHUNK]],
            asem[b]).start(add=True))

    def swait_a(j, b):
        pltpu.make_async_copy(bufs[b], acca.at[didx[0].at[0]], asem[b]).wait()

    def scat_b(j, sb):
        _bi(j, lambda p: pltpu.make_async_copy(
            sbufs[sb], accb.at[didx[p].at[j % CHUNK]],
            bsem[sb]).start(add=True))

    def swait_b(j, sb):
        pltpu.make_async_copy(sbufs[sb], accb.at[didx[0].at[0]],
                              bsem[sb]).wait()

    def scale(j, b, sb):
        def stage(p):
            for g in range(K // LANES):
                sl = pl.ds(g * LANES, LANES)
                sstage[sl] = sidx[p][j % CHUNK, sl] >> 1

        _bi(j, stage)
        for g in range(K // LANES):
            idxv = sstage[pl.ds(g * LANES, LANES)]
            rv = plsc.load_gather(r2v, [idxv])
            for i in range(LANES):
                row = g * LANES + i
                rs = rv[i]
                for q in range(DH // LANES):
                    sl = pl.ds(q * LANES, LANES)
                    sbufs[sb][row, sl] = rs * bufs[b][row, sl]

    gather(0, 0)
    gather(1, 1)

    nfull = BPT // NBUF

    def body(jj, _):
        for b in range(NBUF):
            j = NBUF * jj + b
            sb = b % NSB
            jm = j % CHUNK

            @pl.when(jnp.logical_and(jm == 6, j + (CHUNK - 6) < BPT))
            def _():
                q1 = j // CHUNK + 1

                @pl.when(q1 % 2 == 0)
                def _():
                    load_chunk(q1, 0)

                @pl.when(q1 % 2 == 1)
                def _():
                    load_chunk(q1, 1)

            @pl.when(jm == CHUNK - 2)
            def _():
                q1 = j // CHUNK + 1

                @pl.when(q1 % 2 == 0)
                def _():
                    wait_chunk(0)

                @pl.when(q1 % 2 == 1)
                def _():
                    wait_chunk(1)

            gwait(j, b)
            scat_a(j, b)
            bn = (b + 2) % NBUF

            @pl.when(j + 2 < BPT)
            def _():
                @pl.when(j >= 2)
                def _():
                    swait_a(j - 2, bn)

                gather(j + 2, bn)

            @pl.when(j >= NSB)
            def _():
                swait_b(j - NSB, sb)

            scale(j, b, sb)
            scat_b(j, sb)

        return 0

    lax.fori_loop(0, nfull, body, 0)
    for t in range(nfull * NBUF, BPT):
        b = t % NBUF
        sb = t % NSB
        gwait(t, b)
        scat_a(t, b)
        swait_b(t - NSB, sb)
        scale(t, b, sb)
        scat_b(t, sb)
    for t in range(BPT - NBUF, BPT):
        swait_a(t, t % NBUF)
    for t in range(BPT - NSB, BPT):
        swait_b(t, t % NSB)

    plsc.subcore_barrier()
    r0 = s * RPT
    pltpu.make_async_copy(acca.at[pl.ds(r0, RPT)],
                          ab_out.at[0].at[c].at[pl.ds(r0, RPT)],
                          gsem[0]).start()
    pltpu.make_async_copy(accb.at[pl.ds(r0, RPT)],
                          ab_out.at[1].at[c].at[pl.ds(r0, RPT)],
                          gsem[1]).start()
    pltpu.make_async_copy(acca.at[pl.ds(r0, RPT)],
                          ab_out.at[0].at[c].at[pl.ds(r0, RPT)],
                          gsem[0]).wait()
    pltpu.make_async_copy(accb.at[pl.ds(r0, RPT)],
                          ab_out.at[1].at[c].at[pl.ds(r0, RPT)],
                          gsem[1]).wait()


def _sc_agg(tab2, r2, src0_rs, src1_rs, dst_rs):
    mesh = plsc.VectorSubcoreMesh(core_axis_name="c", subcore_axis_name="s")
    f = pl.kernel(
        _sc_agg_body,
        out_type=jax.ShapeDtypeStruct((2, NC, NPAD, DH), jnp.float32),
        mesh=mesh,
        scratch_types=(
            [pltpu.VMEM((CHUNK, K), jnp.int32)] * 2
            + [pltpu.VMEM((CHUNK, K), jnp.int32)] * 2
            + [pltpu.VMEM((K, DH), jnp.float32)] * NBUF
            + [pltpu.VMEM((K, DH), jnp.float32)] * NSB
            + [pltpu.VMEM((K,), jnp.int32),
               pltpu.VMEM((N,), jnp.float32),
               pltpu.VMEM_SHARED((NPAD, DH), jnp.float32),
               pltpu.VMEM_SHARED((NPAD, DH), jnp.float32)]
            + [pltpu.SemaphoreType.DMA] * (2 * NBUF + NSB + 2)
        ),
        compiler_params=pltpu.CompilerParams(needs_layout_passes=False,
                                             use_tc_tiling_on_sc=False),
    )
    return f(tab2, r2, src0_rs, src1_rs, dst_rs)


ROWS_TC = 1000


def _tc_prep_body(hists_ref, r2_ref):
    d = jnp.sum(hists_ref[...], axis=1, keepdims=True)
    d = jnp.maximum(d, 1.0)
    r2_ref[...] = lax.rsqrt(d)


def _tc_prep(hists_t):
    return pl.pallas_call(
        _tc_prep_body,
        grid=(N // ROWS_TC,),
        in_specs=[
            pl.BlockSpec((ROWS_TC, NC * NS), lambda i: (i, 0)),
        ],
        out_specs=pl.BlockSpec((ROWS_TC, 1), lambda i: (i, 0)),
        out_shape=jax.ShapeDtypeStruct((N, 1), jnp.float32),
    )(hists_t)


def _tc_final_body(w_ref, hists_ref, x_ref, alo_ref, ahi_ref, blo_ref,
                   bhi_ref, wgcn_ref, wss_ref, wsn_ref, wgin_ref, wlin_ref,
                   out_ref):
    w0 = w_ref[0]
    w1 = w_ref[1]
    w2 = w_ref[2]
    w3 = w_ref[3]
    d = jnp.sum(hists_ref[...], axis=1, keepdims=True)
    d = jnp.maximum(d, 1.0)
    r = lax.rsqrt(d)
    inv = 1.0 / d
    wmix = w1 * wss_ref[...] + w2 * wgin_ref[...] + w3 * wlin_ref[...]
    x = x_ref[...]
    a = jnp.concatenate([alo_ref[0, 0], ahi_ref[0, 0]], axis=1)
    b = jnp.concatenate([blo_ref[0, 0], bhi_ref[0, 0]], axis=1)
    acc = jnp.dot(x, wmix, preferred_element_type=jnp.float32)
    acc += w2 * jnp.dot(a, wgin_ref[...], preferred_element_type=jnp.float32)
    acc += (w1 * inv) * jnp.dot(a, wsn_ref[...],
                                preferred_element_type=jnp.float32)
    acc += (w0 * r) * jnp.dot(b, wgcn_ref[...],
                              preferred_element_type=jnp.float32)
    out_ref[...] = acc


def _tc_final(weights, hists_t, x, ab, wgcn, wss, wsn, wgin, wlin):
    row_spec = pl.BlockSpec((ROWS_TC, D), lambda i: (i, 0))
    w_spec = pl.BlockSpec((D, D), lambda i: (0, 0))

    def quarter(m, c):
        return pl.BlockSpec((1, 1, ROWS_TC, DH), lambda i, m=m, c=c: (m, c, i, 0))

    return pl.pallas_call(
        _tc_final_body,
        grid=(N // ROWS_TC,),
        in_specs=[
            pl.BlockSpec(memory_space=pltpu.SMEM),
            pl.BlockSpec((ROWS_TC, NC * NS), lambda i: (i, 0)),
            row_spec,
            quarter(0, 0), quarter(0, 1), quarter(1, 0), quarter(1, 1),
            w_spec, w_spec, w_spec, w_spec, w_spec,
        ],
        out_specs=row_spec,
        out_shape=jax.ShapeDtypeStruct((N, D), jnp.float32),
    )(weights, hists_t, x, ab, ab, ab, ab, wgcn, wss, wsn, wgin, wlin)


def kernel(x, edge_index, weights, W_gcn, W_sage_self, W_sage_neigh, W_gin,
           W_lin):
    src = edge_index[0]
    dst = edge_index[1]
    src2 = src * 2
    src0_rs = src2.reshape(BLOCKS, K)
    src1_rs = (src2 + 1).reshape(BLOCKS, K)
    dst_rs = dst.reshape(BLOCKS, K)
    x2 = x.reshape(2 * N, DH)

    hists = _sc_deg(dst)
    hists_t = hists.T
    r2 = _tc_prep(hists_t).reshape(N)
    ab = _sc_agg(x2, r2, src0_rs, src1_rs, dst_rs)
    return _tc_final(weights, hists_t, x, ab,
                     W_gcn, W_sage_self, W_sage_neigh, W_gin, W_lin)
